# baseline (speedup 1.0000x reference)
import functools

import jax
import jax.numpy as jnp
from jax import lax
from jax.experimental import pallas as pl
from jax.experimental.pallas import tpu as pltpu
from jax.experimental.pallas import tpu_sc as plsc


@functools.lru_cache(maxsize=None)
def _make_pool(B, L, V, E):
    info = plsc.get_sparse_core_info()
    NC, NS = info.num_cores, info.num_subcores
    NW = NC * NS
    SPW = B // NW
    C0 = 104
    C1 = L - C0
    NV = E // 16

    mesh = plsc.VectorSubcoreMesh(core_axis_name="c", subcore_axis_name="s")

    @functools.partial(
        pl.kernel,
        mesh=mesh,
        out_type=jax.ShapeDtypeStruct((B, E), jnp.float32),
        compiler_params=pltpu.CompilerParams(use_tc_tiling_on_sc=False),
        scratch_types=[
            pltpu.VMEM((SPW * L,), jnp.int32),
            pltpu.VMEM((2, L, E), jnp.float32),
            pltpu.VMEM((SPW, E), jnp.float32),
            pltpu.SemaphoreType.DMA,
            pltpu.SemaphoreType.DMA,
        ],
    )
    def pool(x_hbm, table_hbm, out_hbm, idx_v, rows_v, out_v, sem0, sem1):
        wid = lax.axis_index("s") * NC + lax.axis_index("c")
        sent_base = wid * SPW
        pltpu.sync_copy(x_hbm.at[pl.ds(sent_base * L, SPW * L)], idx_v)

        def start(s, buf, sem):
            off = s * L
            pltpu.async_copy(
                table_hbm.at[idx_v.at[pl.ds(off, C0)]],
                rows_v.at[buf, pl.ds(0, C0), :], sem)
            pltpu.async_copy(
                table_hbm.at[idx_v.at[pl.ds(off + C0, C1)]],
                rows_v.at[buf, pl.ds(C0, C1), :], sem)

        def wait(buf, sem):
            pltpu.make_async_copy(
                table_hbm.at[idx_v.at[pl.ds(0, C0)]],
                rows_v.at[buf, pl.ds(0, C0), :], sem).wait()
            pltpu.make_async_copy(
                table_hbm.at[idx_v.at[pl.ds(0, C1)]],
                rows_v.at[buf, pl.ds(C0, C1), :], sem).wait()

        def reduce_store(s, buf):
            def rbody(j, acc):
                return tuple(
                    acc[k] + rows_v[buf, j, pl.ds(16 * k, 16)]
                    for k in range(NV))
            z = jnp.zeros((16,), jnp.float32)
            acc = lax.fori_loop(0, L, rbody, (z,) * NV, unroll=4)
            scale = jnp.float32(1.0 / L)
            for k in range(NV):
                out_v[s, pl.ds(16 * k, 16)] = acc[k] * scale

        start(0, 0, sem0)
        start(1, 1, sem1)

        def body(i, _):
            s0 = 2 * i
            wait(0, sem0)
            reduce_store(s0, 0)

            @pl.when(s0 + 2 < SPW)
            def _():
                start(s0 + 2, 0, sem0)

            wait(1, sem1)
            reduce_store(s0 + 1, 1)

            @pl.when(s0 + 3 < SPW)
            def _():
                start(s0 + 3, 1, sem1)

            return 0

        lax.fori_loop(0, SPW // 2, body, 0)
        pltpu.sync_copy(out_v, out_hbm.at[pl.ds(sent_base, SPW), :])

    return pool


def _mlp_body(pooled_ref, w1_ref, b1_ref, w2_ref, b2_ref, out_ref):
    p = pooled_ref[...]
    h = jnp.dot(p, w1_ref[...], preferred_element_type=jnp.float32) + b1_ref[...]
    z = jnp.dot(h, w2_ref[...], preferred_element_type=jnp.float32) + b2_ref[...]
    z = z - jnp.max(z, axis=1, keepdims=True)
    e = jnp.exp(z)
    out_ref[...] = e / jnp.sum(e, axis=1, keepdims=True)


def kernel(x, emb_table, W1, b1, W2, b2):
    B, L = x.shape
    V, E = emb_table.shape
    pooled = _make_pool(B, L, V, E)(x.reshape(B * L), emb_table)
    mlp = pl.pallas_call(
        _mlp_body,
        out_shape=jax.ShapeDtypeStruct((B, W2.shape[1]), jnp.float32),
    )
    return mlp(pooled, W1, b1.reshape(1, -1), W2, b2.reshape(1, -1))


# speedup vs baseline: 1.0022x; 1.0022x over previous
import functools

import jax
import jax.numpy as jnp
from jax import lax
from jax.experimental import pallas as pl
from jax.experimental.pallas import tpu as pltpu
from jax.experimental.pallas import tpu_sc as plsc


@functools.lru_cache(maxsize=None)
def _make_pool(B, L, V, E):
    info = plsc.get_sparse_core_info()
    NC, NS = info.num_cores, info.num_subcores
    NW = NC * NS
    SPW = B // NW
    C0 = 104
    C1 = L - C0
    NV = E // 16

    mesh = plsc.VectorSubcoreMesh(core_axis_name="c", subcore_axis_name="s")

    @functools.partial(
        pl.kernel,
        mesh=mesh,
        out_type=jax.ShapeDtypeStruct((B, E), jnp.float32),
        compiler_params=pltpu.CompilerParams(use_tc_tiling_on_sc=False),
        scratch_types=[
            pltpu.VMEM((SPW, L), jnp.int32),
            pltpu.VMEM((2, L, E), jnp.float32),
            pltpu.VMEM((SPW, E), jnp.float32),
            pltpu.SemaphoreType.DMA,
            pltpu.SemaphoreType.DMA,
        ],
    )
    def pool(x_hbm, table_hbm, out_hbm, idx_v, rows_v, out_v, sem0, sem1):
        wid = lax.axis_index("s") * NC + lax.axis_index("c")
        sent_base = wid * SPW
        pltpu.sync_copy(x_hbm.at[pl.ds(sent_base, SPW), :], idx_v)

        def start(s, buf, sem):
            pltpu.async_copy(
                table_hbm.at[idx_v.at[s, pl.ds(0, C0)]],
                rows_v.at[buf, pl.ds(0, C0), :], sem)
            pltpu.async_copy(
                table_hbm.at[idx_v.at[s, pl.ds(C0, C1)]],
                rows_v.at[buf, pl.ds(C0, C1), :], sem)

        def wait(buf, sem):
            pltpu.make_async_copy(
                table_hbm.at[idx_v.at[0, pl.ds(0, C0)]],
                rows_v.at[buf, pl.ds(0, C0), :], sem).wait()
            pltpu.make_async_copy(
                table_hbm.at[idx_v.at[0, pl.ds(C0, C1)]],
                rows_v.at[buf, pl.ds(C0, C1), :], sem).wait()

        def reduce_store(s, buf):
            def rbody(j, acc):
                return tuple(
                    acc[k] + rows_v[buf, j, pl.ds(16 * k, 16)]
                    for k in range(NV))
            z = jnp.zeros((16,), jnp.float32)
            acc = lax.fori_loop(0, L, rbody, (z,) * NV, unroll=4)
            scale = jnp.float32(1.0 / L)
            for k in range(NV):
                out_v[s, pl.ds(16 * k, 16)] = acc[k] * scale

        start(0, 0, sem0)
        start(1, 1, sem1)

        def body(i, _):
            s0 = 2 * i
            wait(0, sem0)
            reduce_store(s0, 0)

            @pl.when(s0 + 2 < SPW)
            def _():
                start(s0 + 2, 0, sem0)

            wait(1, sem1)
            reduce_store(s0 + 1, 1)

            @pl.when(s0 + 3 < SPW)
            def _():
                start(s0 + 3, 1, sem1)

            return 0

        lax.fori_loop(0, SPW // 2, body, 0)
        pltpu.sync_copy(out_v, out_hbm.at[pl.ds(sent_base, SPW), :])

    return pool


def _mlp_body(pooled_ref, w1_ref, b1_ref, w2_ref, b2_ref, out_ref):
    p = pooled_ref[...]
    h = jnp.dot(p, w1_ref[...], preferred_element_type=jnp.float32) + b1_ref[...]
    z = jnp.dot(h, w2_ref[...], preferred_element_type=jnp.float32) + b2_ref[...]
    z = z - jnp.max(z, axis=1, keepdims=True)
    e = jnp.exp(z)
    out_ref[...] = e / jnp.sum(e, axis=1, keepdims=True)


def kernel(x, emb_table, W1, b1, W2, b2):
    B, L = x.shape
    V, E = emb_table.shape
    pooled = _make_pool(B, L, V, E)(x, emb_table)
    mlp = pl.pallas_call(
        _mlp_body,
        out_shape=jax.ShapeDtypeStruct((B, W2.shape[1]), jnp.float32),
    )
    return mlp(pooled, W1, b1.reshape(1, -1), W2, b2.reshape(1, -1))


# speedup vs baseline: 1.4948x; 1.4915x over previous
import functools

import jax
import jax.numpy as jnp
from jax import lax
from jax.experimental import pallas as pl
from jax.experimental.pallas import tpu as pltpu
from jax.experimental.pallas import tpu_sc as plsc


@functools.lru_cache(maxsize=None)
def _make_pool(B, L, V, E):
    info = plsc.get_sparse_core_info()
    NC, NS = info.num_cores, info.num_subcores
    NW = NC * NS
    SPW = B // NW
    C0 = 104
    C1 = L - C0
    NV = E // 16
    EP = 2 * E

    mesh = plsc.VectorSubcoreMesh(core_axis_name="c", subcore_axis_name="s")

    @functools.partial(
        pl.kernel,
        mesh=mesh,
        out_type=jax.ShapeDtypeStruct((B, E), jnp.float32),
        compiler_params=pltpu.CompilerParams(use_tc_tiling_on_sc=True),
        scratch_types=[
            pltpu.VMEM((SPW * L,), jnp.int32),
            pltpu.VMEM((2, L, EP), jnp.float32),
            pltpu.VMEM((SPW, E), jnp.float32),
            pltpu.SemaphoreType.DMA,
            pltpu.SemaphoreType.DMA,
        ],
    )
    def pool(x_hbm, table_hbm, out_hbm, idx_v, rows_v, out_v, sem0, sem1):
        wid = lax.axis_index("s") * NC + lax.axis_index("c")
        sent_base = wid * SPW
        pltpu.sync_copy(x_hbm.at[pl.ds(sent_base * L, SPW * L)], idx_v)

        def start(s, buf, sem):
            off = s * L
            pltpu.async_copy(
                table_hbm.at[idx_v.at[pl.ds(off, C0)]],
                rows_v.at[buf, pl.ds(0, C0), :], sem)
            pltpu.async_copy(
                table_hbm.at[idx_v.at[pl.ds(off + C0, C1)]],
                rows_v.at[buf, pl.ds(C0, C1), :], sem)

        def wait(buf, sem):
            pltpu.make_async_copy(
                table_hbm.at[idx_v.at[pl.ds(0, C0)]],
                rows_v.at[buf, pl.ds(0, C0), :], sem).wait()
            pltpu.make_async_copy(
                table_hbm.at[idx_v.at[pl.ds(0, C1)]],
                rows_v.at[buf, pl.ds(C0, C1), :], sem).wait()

        def reduce_store(s, buf):
            def rbody(j, acc):
                return tuple(
                    acc[k] + rows_v[buf, j, pl.ds(16 * k, 16)]
                    for k in range(NV))
            z = jnp.zeros((16,), jnp.float32)
            acc = lax.fori_loop(0, L, rbody, (z,) * NV, unroll=4)
            scale = jnp.float32(1.0 / L)
            for k in range(NV):
                out_v[s, pl.ds(16 * k, 16)] = acc[k] * scale

        start(0, 0, sem0)
        start(1, 1, sem1)

        def body(i, _):
            s0 = 2 * i
            wait(0, sem0)
            reduce_store(s0, 0)

            @pl.when(s0 + 2 < SPW)
            def _():
                start(s0 + 2, 0, sem0)

            wait(1, sem1)
            reduce_store(s0 + 1, 1)

            @pl.when(s0 + 3 < SPW)
            def _():
                start(s0 + 3, 1, sem1)

            return 0

        lax.fori_loop(0, SPW // 2, body, 0)
        pltpu.sync_copy(out_v, out_hbm.at[pl.ds(sent_base, SPW), :])

    return pool


def _repack_body(xt_ref, out_ref):
    E = xt_ref.shape[0]
    out_ref[:, :E] = xt_ref[...].T


@functools.lru_cache(maxsize=None)
def _make_repack(V, E, W=8192):
    grid = (V + W - 1) // W
    return pl.pallas_call(
        _repack_body,
        grid=(grid,),
        in_specs=[pl.BlockSpec((E, W), lambda i: (0, i))],
        out_specs=pl.BlockSpec((W, 2 * E), lambda i: (i, 0)),
        out_shape=jax.ShapeDtypeStruct((V, 2 * E), jnp.float32),
    )


def _mlp_body(pooled_ref, w1_ref, b1_ref, w2_ref, b2_ref, out_ref):
    p = pooled_ref[...]
    h = jnp.dot(p, w1_ref[...], preferred_element_type=jnp.float32) + b1_ref[...]
    z = jnp.dot(h, w2_ref[...], preferred_element_type=jnp.float32) + b2_ref[...]
    z = z - jnp.max(z, axis=1, keepdims=True)
    e = jnp.exp(z)
    out_ref[...] = e / jnp.sum(e, axis=1, keepdims=True)


def kernel(x, emb_table, W1, b1, W2, b2):
    B, L = x.shape
    V, E = emb_table.shape
    packed = _make_repack(V, E)(emb_table.T)
    pooled = _make_pool(B, L, V, E)(x.reshape(B * L), packed)
    mlp = pl.pallas_call(
        _mlp_body,
        out_shape=jax.ShapeDtypeStruct((B, W2.shape[1]), jnp.float32),
    )
    return mlp(pooled, W1, b1.reshape(1, -1), W2, b2.reshape(1, -1))


# speedup vs baseline: 1.5577x; 1.0421x over previous
import functools

import jax
import jax.numpy as jnp
from jax import lax
from jax.experimental import pallas as pl
from jax.experimental.pallas import tpu as pltpu
from jax.experimental.pallas import tpu_sc as plsc


@functools.lru_cache(maxsize=None)
def _make_pool(B, L, V, E):
    info = plsc.get_sparse_core_info()
    NC, NS = info.num_cores, info.num_subcores
    NW = NC * NS
    SPW = B // NW
    C0 = 104
    C1 = L - C0
    NV = E // 16
    EP = 2 * E

    mesh = plsc.VectorSubcoreMesh(core_axis_name="c", subcore_axis_name="s")

    @functools.partial(
        pl.kernel,
        mesh=mesh,
        out_type=jax.ShapeDtypeStruct((B, E), jnp.float32),
        compiler_params=pltpu.CompilerParams(use_tc_tiling_on_sc=True),
        scratch_types=[
            pltpu.VMEM((SPW * L,), jnp.int32),
            pltpu.VMEM((2, L, EP), jnp.float32),
            pltpu.VMEM((SPW, E), jnp.float32),
            pltpu.SemaphoreType.DMA,
            pltpu.SemaphoreType.DMA,
        ],
    )
    def pool(x_hbm, table_hbm, out_hbm, idx_v, rows_v, out_v, sem0, sem1):
        wid = lax.axis_index("s") * NC + lax.axis_index("c")
        sent_base = wid * SPW
        pltpu.sync_copy(x_hbm.at[pl.ds(sent_base * L, SPW * L)], idx_v)

        def start(s, buf, sem):
            off = s * L
            pltpu.async_copy(
                table_hbm.at[idx_v.at[pl.ds(off, C0)]],
                rows_v.at[buf, pl.ds(0, C0), :], sem)
            pltpu.async_copy(
                table_hbm.at[idx_v.at[pl.ds(off + C0, C1)]],
                rows_v.at[buf, pl.ds(C0, C1), :], sem)

        def wait(buf, sem):
            pltpu.make_async_copy(
                table_hbm.at[idx_v.at[pl.ds(0, C0)]],
                rows_v.at[buf, pl.ds(0, C0), :], sem).wait()
            pltpu.make_async_copy(
                table_hbm.at[idx_v.at[pl.ds(0, C1)]],
                rows_v.at[buf, pl.ds(C0, C1), :], sem).wait()

        def reduce_store(s, buf):
            def rbody(j, acc):
                return tuple(
                    acc[k] + rows_v[buf, j, pl.ds(16 * k, 16)]
                    for k in range(NV))
            z = jnp.zeros((16,), jnp.float32)
            acc = lax.fori_loop(0, L, rbody, (z,) * NV, unroll=4)
            scale = jnp.float32(1.0 / L)
            for k in range(NV):
                out_v[s, pl.ds(16 * k, 16)] = acc[k] * scale

        start(0, 0, sem0)
        start(1, 1, sem1)

        def body(i, _):
            s0 = 2 * i
            wait(0, sem0)
            reduce_store(s0, 0)

            @pl.when(s0 + 2 < SPW)
            def _():
                start(s0 + 2, 0, sem0)

            wait(1, sem1)
            reduce_store(s0 + 1, 1)

            @pl.when(s0 + 3 < SPW)
            def _():
                start(s0 + 3, 1, sem1)

            return 0

        lax.fori_loop(0, SPW // 2, body, 0)
        pltpu.sync_copy(out_v, out_hbm.at[pl.ds(sent_base, SPW), :])

    return pool


def _repack_body(xt_ref, out_ref):
    E = xt_ref.shape[0]
    out_ref[:, :E] = xt_ref[...].T


@functools.lru_cache(maxsize=None)
def _make_repack(V, E, W=16384):
    grid = (V + W - 1) // W
    return pl.pallas_call(
        _repack_body,
        grid=(grid,),
        in_specs=[pl.BlockSpec((E, W), lambda i: (0, i))],
        out_specs=pl.BlockSpec((W, 2 * E), lambda i: (i, 0)),
        out_shape=jax.ShapeDtypeStruct((V, 2 * E), jnp.float32),
    )


def _mlp_body(pooled_ref, w1_ref, b1_ref, w2_ref, b2_ref, out_ref):
    p = pooled_ref[...]
    h = jnp.dot(p, w1_ref[...], preferred_element_type=jnp.float32) + b1_ref[...]
    z = jnp.dot(h, w2_ref[...], preferred_element_type=jnp.float32) + b2_ref[...]
    z = z - jnp.max(z, axis=1, keepdims=True)
    e = jnp.exp(z)
    out_ref[...] = e / jnp.sum(e, axis=1, keepdims=True)


def kernel(x, emb_table, W1, b1, W2, b2):
    B, L = x.shape
    V, E = emb_table.shape
    packed = _make_repack(V, E)(emb_table.T)
    pooled = _make_pool(B, L, V, E)(x.reshape(B * L), packed)
    mlp = pl.pallas_call(
        _mlp_body,
        out_shape=jax.ShapeDtypeStruct((B, W2.shape[1]), jnp.float32),
    )
    return mlp(pooled, W1, b1.reshape(1, -1), W2, b2.reshape(1, -1))


# speedup vs baseline: 1.5818x; 1.0154x over previous
import functools

import jax
import jax.numpy as jnp
from jax import lax
from jax.experimental import pallas as pl
from jax.experimental.pallas import tpu as pltpu
from jax.experimental.pallas import tpu_sc as plsc


@functools.lru_cache(maxsize=None)
def _make_pool(B, L, V, E):
    info = plsc.get_sparse_core_info()
    NC, NS = info.num_cores, info.num_subcores
    NW = NC * NS
    SPW = B // NW
    C0 = 104
    C1 = L - C0
    NV = E // 16
    EP = 2 * E

    mesh = plsc.VectorSubcoreMesh(core_axis_name="c", subcore_axis_name="s")

    @functools.partial(
        pl.kernel,
        mesh=mesh,
        out_type=jax.ShapeDtypeStruct((B, E), jnp.float32),
        compiler_params=pltpu.CompilerParams(use_tc_tiling_on_sc=True),
        scratch_types=[
            pltpu.VMEM((SPW * L,), jnp.int32),
            pltpu.VMEM((2, L, EP), jnp.float32),
            pltpu.VMEM((SPW, E), jnp.float32),
            pltpu.SemaphoreType.DMA,
            pltpu.SemaphoreType.DMA,
        ],
    )
    def pool(x_hbm, table_hbm, out_hbm, idx_v, rows_v, out_v, sem0, sem1):
        wid = lax.axis_index("s") * NC + lax.axis_index("c")
        sent_base = wid * SPW
        pltpu.sync_copy(x_hbm.at[pl.ds(sent_base * L, SPW * L)], idx_v)

        def start(s, buf, sem):
            off = s * L
            pltpu.async_copy(
                table_hbm.at[idx_v.at[pl.ds(off, C0)]],
                rows_v.at[buf, pl.ds(0, C0), :], sem)
            pltpu.async_copy(
                table_hbm.at[idx_v.at[pl.ds(off + C0, C1)]],
                rows_v.at[buf, pl.ds(C0, C1), :], sem)

        def wait(buf, sem):
            pltpu.make_async_copy(
                table_hbm.at[idx_v.at[pl.ds(0, C0)]],
                rows_v.at[buf, pl.ds(0, C0), :], sem).wait()
            pltpu.make_async_copy(
                table_hbm.at[idx_v.at[pl.ds(0, C1)]],
                rows_v.at[buf, pl.ds(C0, C1), :], sem).wait()

        def reduce_store(s, buf):
            def rbody(j, acc):
                return tuple(
                    acc[k] + rows_v[buf, j, pl.ds(16 * k, 16)]
                    for k in range(NV))
            z = jnp.zeros((16,), jnp.float32)
            acc = lax.fori_loop(0, L, rbody, (z,) * NV, unroll=4)
            scale = jnp.float32(1.0 / L)
            for k in range(NV):
                out_v[s, pl.ds(16 * k, 16)] = acc[k] * scale

        start(0, 0, sem0)
        start(1, 1, sem1)

        def body(i, _):
            s0 = 2 * i
            wait(0, sem0)
            reduce_store(s0, 0)

            @pl.when(s0 + 2 < SPW)
            def _():
                start(s0 + 2, 0, sem0)

            wait(1, sem1)
            reduce_store(s0 + 1, 1)

            @pl.when(s0 + 3 < SPW)
            def _():
                start(s0 + 3, 1, sem1)

            return 0

        lax.fori_loop(0, SPW // 2, body, 0)
        pltpu.sync_copy(out_v, out_hbm.at[pl.ds(sent_base, SPW), :])

    return pool


def _repack_body(xt_ref, out_ref):
    E = xt_ref.shape[0]
    out_ref[:, :E] = xt_ref[...].T


@functools.lru_cache(maxsize=None)
def _make_repack(V, E, W=32768):
    grid = (V + W - 1) // W
    return pl.pallas_call(
        _repack_body,
        grid=(grid,),
        in_specs=[pl.BlockSpec((E, W), lambda i: (0, i))],
        out_specs=pl.BlockSpec((W, 2 * E), lambda i: (i, 0)),
        out_shape=jax.ShapeDtypeStruct((V, 2 * E), jnp.float32),
    )


def _mlp_body(pooled_ref, w1_ref, b1_ref, w2_ref, b2_ref, out_ref):
    p = pooled_ref[...]
    h = jnp.dot(p, w1_ref[...], preferred_element_type=jnp.float32) + b1_ref[...]
    z = jnp.dot(h, w2_ref[...], preferred_element_type=jnp.float32) + b2_ref[...]
    z = z - jnp.max(z, axis=1, keepdims=True)
    e = jnp.exp(z)
    out_ref[...] = e / jnp.sum(e, axis=1, keepdims=True)


def kernel(x, emb_table, W1, b1, W2, b2):
    B, L = x.shape
    V, E = emb_table.shape
    packed = _make_repack(V, E)(emb_table.T)
    pooled = _make_pool(B, L, V, E)(x.reshape(B * L), packed)
    mlp = pl.pallas_call(
        _mlp_body,
        out_shape=jax.ShapeDtypeStruct((B, W2.shape[1]), jnp.float32),
    )
    return mlp(pooled, W1, b1.reshape(1, -1), W2, b2.reshape(1, -1))


# speedup vs baseline: 1.6597x; 1.0492x over previous
import functools

import jax
import jax.numpy as jnp
from jax import lax
from jax.experimental import pallas as pl
from jax.experimental.pallas import tpu as pltpu
from jax.experimental.pallas import tpu_sc as plsc


@functools.lru_cache(maxsize=None)
def _make_pool(B, L, V, E):
    info = plsc.get_sparse_core_info()
    NC, NS = info.num_cores, info.num_subcores
    NW = NC * NS
    SPW = B // NW
    C0 = 104
    C1 = L - C0
    NV = E // 16
    EP = 2 * E

    mesh = plsc.VectorSubcoreMesh(core_axis_name="c", subcore_axis_name="s")

    @functools.partial(
        pl.kernel,
        mesh=mesh,
        out_type=jax.ShapeDtypeStruct((B, E), jnp.float32),
        compiler_params=pltpu.CompilerParams(use_tc_tiling_on_sc=True),
        scratch_types=[
            pltpu.VMEM((SPW * L,), jnp.int32),
            pltpu.VMEM((3, L, EP), jnp.float32),
            pltpu.VMEM((SPW, E), jnp.float32),
            pltpu.SemaphoreType.DMA,
            pltpu.SemaphoreType.DMA,
            pltpu.SemaphoreType.DMA,
        ],
    )
    def pool(x_hbm, table_hbm, out_hbm, idx_v, rows_v, out_v, sem0, sem1, sem2):
        wid = lax.axis_index("s") * NC + lax.axis_index("c")
        sent_base = wid * SPW
        pltpu.sync_copy(x_hbm.at[pl.ds(sent_base * L, SPW * L)], idx_v)

        def start(s, buf, sem):
            off = s * L
            pltpu.async_copy(
                table_hbm.at[idx_v.at[pl.ds(off, C0)]],
                rows_v.at[buf, pl.ds(0, C0), :], sem)
            pltpu.async_copy(
                table_hbm.at[idx_v.at[pl.ds(off + C0, C1)]],
                rows_v.at[buf, pl.ds(C0, C1), :], sem)

        def wait(buf, sem):
            pltpu.make_async_copy(
                table_hbm.at[idx_v.at[pl.ds(0, C0)]],
                rows_v.at[buf, pl.ds(0, C0), :], sem).wait()
            pltpu.make_async_copy(
                table_hbm.at[idx_v.at[pl.ds(0, C1)]],
                rows_v.at[buf, pl.ds(C0, C1), :], sem).wait()

        def reduce_store(s, buf):
            def rbody(j, acc):
                return tuple(
                    acc[k] + rows_v[buf, j, pl.ds(16 * k, 16)]
                    for k in range(NV))
            z = jnp.zeros((16,), jnp.float32)
            acc = lax.fori_loop(0, L, rbody, (z,) * NV, unroll=4)
            scale = jnp.float32(1.0 / L)
            for k in range(NV):
                out_v[s, pl.ds(16 * k, 16)] = acc[k] * scale

        sems = (sem0, sem1, sem2)
        for b in range(3):
            start(b, b, sems[b])

        def body(i, _):
            for b in range(3):
                s = 3 * i + b
                wait(b, sems[b])
                reduce_store(s, b)

                @pl.when(s + 3 < SPW)
                def _():
                    start(s + 3, b, sems[b])

            return 0

        lax.fori_loop(0, SPW // 3, body, 0)
        for b in range(SPW % 3):
            s = (SPW // 3) * 3 + b
            wait(b, sems[b])
            reduce_store(s, b)
        pltpu.sync_copy(out_v, out_hbm.at[pl.ds(sent_base, SPW), :])

    return pool


def _repack_body(xt_ref, out_ref):
    E = xt_ref.shape[0]
    out_ref[:, :E] = xt_ref[...].T


@functools.lru_cache(maxsize=None)
def _make_repack(V, E, W=32768):
    grid = (V + W - 1) // W
    return pl.pallas_call(
        _repack_body,
        grid=(grid,),
        in_specs=[pl.BlockSpec((E, W), lambda i: (0, i))],
        out_specs=pl.BlockSpec((W, 2 * E), lambda i: (i, 0)),
        out_shape=jax.ShapeDtypeStruct((V, 2 * E), jnp.float32),
    )


def _mlp_body(pooled_ref, w1_ref, b1_ref, w2_ref, b2_ref, out_ref):
    p = pooled_ref[...]
    h = jnp.dot(p, w1_ref[...], preferred_element_type=jnp.float32) + b1_ref[...]
    z = jnp.dot(h, w2_ref[...], preferred_element_type=jnp.float32) + b2_ref[...]
    z = z - jnp.max(z, axis=1, keepdims=True)
    e = jnp.exp(z)
    out_ref[...] = e / jnp.sum(e, axis=1, keepdims=True)


def kernel(x, emb_table, W1, b1, W2, b2):
    B, L = x.shape
    V, E = emb_table.shape
    packed = _make_repack(V, E)(emb_table.T)
    pooled = _make_pool(B, L, V, E)(x.reshape(B * L), packed)
    mlp = pl.pallas_call(
        _mlp_body,
        out_shape=jax.ShapeDtypeStruct((B, W2.shape[1]), jnp.float32),
    )
    return mlp(pooled, W1, b1.reshape(1, -1), W2, b2.reshape(1, -1))
